# baseline (speedup 1.0000x reference)
import functools

import jax
import jax.numpy as jnp
from jax import lax
from jax.experimental import pallas as pl
from jax.experimental.pallas import tpu as pltpu
from jax.experimental.pallas import tpu_sc as plsc

NUM_FREQ = 12
EMB_DIM = 22
N_VARS = 100
CH = 2 * NUM_FREQ + EMB_DIM

_C = (0.9999999995124089, -4.934802118487793, 4.05870883800603,
      -1.3352100152568833, 0.23493326541101656, -0.02439611339077682)


def _sinpi(t):
    ft = jnp.floor(t)
    u = t - ft
    half = ft * 0.5
    par = half - jnp.floor(half)
    sign = 1.0 - 4.0 * par
    w = u - 0.5
    z = w * w
    p = _C[5]
    p = p * z + _C[4]
    p = p * z + _C[3]
    p = p * z + _C[2]
    p = p * z + _C[1]
    p = p * z + _C[0]
    return p * sign


def _geo_kernel(auxT_ref, tbl_ref, out_ref, mask_ref):
    x = auxT_ref[...]
    s = _sinpi(x)
    c = _sinpi(x + 0.5)
    out_ref[0] = s
    out_ref[NUM_FREQ] = c
    for k in range(1, NUM_FREQ):
        s, c = 2.0 * s * c, 1.0 - 2.0 * s * s
        out_ref[k] = s
        out_ref[NUM_FREQ + k] = c
    e = tbl_ref[...]
    out_ref[2 * NUM_FREQ:] = jnp.broadcast_to(e, (EMB_DIM,) + x.shape)
    mask_ref[...] = (x != x).astype(jnp.int8)


_GC = 4
_NG = 2500
_NCHUNK = 20


def _mask_sc(z_hbm, out_hbm, zbuf, sem):
    w = lax.axis_index("s") * 2 + lax.axis_index("c")
    pltpu.async_copy(z_hbm, zbuf, sem).wait()
    gb = (_NG * w) // 32
    descs = []
    for j in range(_NCHUNK):
        g = jnp.minimum(gb + j * _GC, _NG - _GC)
        descs.append(pltpu.async_copy(zbuf, out_hbm.at[pl.ds(g, _GC)], sem))
    for d in descs:
        d.wait()


@functools.partial(jax.jit, static_argnames=())
def kernel(aux_values, predictor_values, table):
    B, V = aux_values.shape
    L = predictor_values.shape[1]
    auxT = aux_values.T
    tblT = table.T.reshape(EMB_DIM, V, 1)

    Bb = 256
    grid = (B // Bb,)
    outT, _ = pl.pallas_call(
        _geo_kernel,
        grid=grid,
        in_specs=[
            pl.BlockSpec((V, Bb), lambda i: (0, i)),
            pl.BlockSpec((EMB_DIM, V, 1), lambda i: (0, 0, 0)),
        ],
        out_specs=[
            pl.BlockSpec((CH, V, Bb), lambda i: (0, 0, i)),
            pl.BlockSpec((V, Bb), lambda i: (0, i)),
        ],
        out_shape=[
            jax.ShapeDtypeStruct((CH, V, B), jnp.float32),
            jax.ShapeDtypeStruct((V, B), jnp.int8),
        ],
    )(auxT, tblT)
    out = outT.transpose(2, 1, 0)

    zrow = jnp.zeros((_GC, 8, B), jnp.bool_)
    mesh = plsc.VectorSubcoreMesh(core_axis_name="c", subcore_axis_name="s")
    maskT = pl.kernel(
        _mask_sc,
        out_type=jax.ShapeDtypeStruct((_NG, 8, B), jnp.bool_),
        mesh=mesh,
        scratch_types=[
            pltpu.VMEM((_GC, 8, B), jnp.bool_),
            pltpu.SemaphoreType.DMA,
        ],
    )(zrow)
    mask = maskT.reshape(V, L, B).transpose(2, 1, 0)
    return (out, mask)


# speedup vs baseline: 5.0941x; 5.0941x over previous
import functools

import jax
import jax.numpy as jnp
from jax.experimental import pallas as pl

NUM_FREQ = 12
EMB_DIM = 22
N_VARS = 100
CH = 2 * NUM_FREQ + EMB_DIM

_C = (0.9999999995124089, -4.934802118487793, 4.05870883800603,
      -1.3352100152568833, 0.23493326541101656, -0.02439611339077682)


def _sinpi(t):
    ft = jnp.floor(t)
    u = t - ft
    half = ft * 0.5
    par = half - jnp.floor(half)
    sign = 1.0 - 4.0 * par
    w = u - 0.5
    z = w * w
    p = _C[5]
    p = p * z + _C[4]
    p = p * z + _C[3]
    p = p * z + _C[2]
    p = p * z + _C[1]
    p = p * z + _C[0]
    return p * sign


def _geo_kernel(auxT_ref, tbl_ref, out_ref, mask_ref):
    x = auxT_ref[...]
    s = _sinpi(x)
    c = _sinpi(x + 0.5)
    out_ref[0] = s
    out_ref[NUM_FREQ] = c
    for k in range(1, NUM_FREQ):
        s, c = 2.0 * s * c, 1.0 - 2.0 * s * s
        out_ref[k] = s
        out_ref[NUM_FREQ + k] = c
    e = tbl_ref[...]
    out_ref[2 * NUM_FREQ:] = jnp.broadcast_to(e, (EMB_DIM,) + x.shape)
    mask_ref[...] = (x != x).astype(jnp.int8)


@functools.partial(jax.jit, static_argnames=())
def kernel(aux_values, predictor_values, table):
    B, V = aux_values.shape
    L = predictor_values.shape[1]
    auxT = aux_values.T
    tblT = table.T.reshape(EMB_DIM, V, 1)

    Bb = 256
    grid = (B // Bb,)
    outT, maskT = pl.pallas_call(
        _geo_kernel,
        grid=grid,
        in_specs=[
            pl.BlockSpec((V, Bb), lambda i: (0, i)),
            pl.BlockSpec((EMB_DIM, V, 1), lambda i: (0, 0, 0)),
        ],
        out_specs=[
            pl.BlockSpec((CH, V, Bb), lambda i: (0, 0, i)),
            pl.BlockSpec((V, Bb), lambda i: (0, i)),
        ],
        out_shape=[
            jax.ShapeDtypeStruct((CH, V, B), jnp.float32),
            jax.ShapeDtypeStruct((V, B), jnp.int8),
        ],
    )(auxT, tblT)
    out = outT.transpose(2, 1, 0)
    mask = jnp.broadcast_to((maskT.T != 0)[:, None, :], (B, L, V))
    return (out, mask)
